# baseline (speedup 1.0000x reference)
import functools

import jax
import jax.numpy as jnp
from jax import lax
from jax.experimental import pallas as pl
from jax.experimental.pallas import tpu as pltpu
from jax.experimental.pallas import tpu_sc as plsc

_N = 10000
_E = 320000
_G = 16
_C = 128
_NC, _NS = 2, 16
_NW = _NC * _NS
_EW = _E // _NW
_B = 80
_NCH = _EW // _B
_NSTG = 5
_WIN = 24
_TCH = _NCH - _NSTG * _WIN
_RPS = 624
_TAIL = _N - _RPS * _NS
_BM = 2000


def _agg_body(h_hbm, src_hbm, dst_hbm, z_hbm, out_hbm,
              src_v, dst_v, r0, r1, r2, r3, table_sh,
              g0, g1, g2, g3, s0, s1, s2, s3):
    c = lax.axis_index("c")
    s = lax.axis_index("s")
    w = c * _NS + s
    bufs = ((r0, g0, s0), (r1, g1, s1), (r2, g2, s2), (r3, g3, s3))
    pltpu.sync_copy(z_hbm.at[pl.ds(s * _RPS, _RPS)],
                    table_sh.at[pl.ds(s * _RPS, _RPS)])

    @pl.when(s == _NS - 1)
    def _():
        pltpu.sync_copy(z_hbm.at[pl.ds(_RPS * _NS, _TAIL)],
                        table_sh.at[pl.ds(_RPS * _NS, _TAIL)])
    plsc.subcore_barrier()

    def G(t, k):
        pltpu.async_copy(h_hbm.at[src_v.at[t]], bufs[k][0], bufs[k][1])

    def WG(k):
        pltpu.make_async_copy(
            h_hbm.at[src_v.at[0]], bufs[k][0], bufs[k][1]).wait()

    def S(t, k):
        pltpu.async_copy(bufs[k][0], table_sh.at[dst_v.at[t]], bufs[k][2],
                         add=True)

    def WS(k):
        pltpu.make_async_copy(
            bufs[k][0], table_sh.at[dst_v.at[0]], bufs[k][2]).wait()

    for stg in range(_NSTG):
        pltpu.sync_copy(src_hbm.at[w, pl.ds(stg * _WIN, _WIN)], src_v)
        pltpu.sync_copy(dst_hbm.at[w, pl.ds(stg * _WIN, _WIN)], dst_v)
        G(0, 0)
        G(1, 1)
        WG(0)
        S(0, 0)
        G(2, 2)
        WG(1)
        S(1, 1)
        G(3, 3)

        def quad(j, carry):
            t0 = 4 * j + 2
            for k in range(4):
                t = t0 + k
                kb = (2 + k) % 4
                WG(kb)
                S(t, kb)
                WS((kb + 2) % 4)
                G(t + 2, (kb + 2) % 4)
            return carry

        lax.fori_loop(0, (_WIN - 4) // 4, quad, 0)
        WG(2)
        S(_WIN - 2, 2)
        WG(3)
        S(_WIN - 1, 3)
        for k in range(4):
            WS(k)

    pltpu.sync_copy(src_hbm.at[w, pl.ds(_NSTG * _WIN, _TCH)],
                    src_v.at[pl.ds(0, _TCH)])
    pltpu.sync_copy(dst_hbm.at[w, pl.ds(_NSTG * _WIN, _TCH)],
                    dst_v.at[pl.ds(0, _TCH)])
    for t in range(_TCH):
        pltpu.async_copy(h_hbm.at[src_v.at[t]], r0, g0).wait()
        pltpu.sync_copy(r0, table_sh.at[dst_v.at[t]], add=True)
    plsc.subcore_barrier()
    pltpu.sync_copy(table_sh.at[pl.ds(s * _RPS, _RPS)],
                    out_hbm.at[c, pl.ds(s * _RPS, _RPS)])

    @pl.when(s == _NS - 1)
    def _():
        pltpu.sync_copy(table_sh.at[pl.ds(_RPS * _NS, _TAIL)],
                        out_hbm.at[c, pl.ds(_RPS * _NS, _TAIL)])


@functools.cache
def _get_agg():
    return pl.kernel(
        _agg_body,
        out_type=jax.ShapeDtypeStruct((_NC, _N, _C), jnp.float32),
        mesh=plsc.VectorSubcoreMesh(core_axis_name="c", subcore_axis_name="s"),
        scratch_types=(
            [pltpu.VMEM((_WIN, _B), jnp.int32)] * 2
            + [pltpu.VMEM((_B, _C), jnp.float32)] * 4
            + [pltpu.VMEM_SHARED((_N, _C), jnp.float32)]
            + [pltpu.SemaphoreType.DMA] * 8
        ),
    )


def _layer_body(relu, split_out, p_ref, x_ref, wrel_ref, wroot_ref, b_ref,
                *o_refs):
    agg = p_ref[0] + p_ref[1]
    acc = jnp.dot(agg, wrel_ref[...], preferred_element_type=jnp.float32)
    acc = acc + jnp.dot(x_ref[...], wroot_ref[...],
                        preferred_element_type=jnp.float32)
    acc = acc + b_ref[...]
    if relu:
        acc = jnp.maximum(acc, 0.0)
    if split_out:
        o_refs[0][...] = acc[:, :_C]
        o_refs[1][...] = acc[:, _C:]
    else:
        o_refs[0][...] = acc


def _make_layer(cout, relu, split_out):
    out_shape = (
        [jax.ShapeDtypeStruct((_N, _C), jnp.float32)] * 2 if split_out
        else jax.ShapeDtypeStruct((_N, cout), jnp.float32))
    out_specs = (
        [pl.BlockSpec((_BM, _C), lambda i: (i, 0))] * 2 if split_out
        else pl.BlockSpec((_BM, cout), lambda i: (i, 0)))
    return pl.pallas_call(
        functools.partial(_layer_body, relu, split_out),
        grid=(_N // _BM,),
        in_specs=[
            pl.BlockSpec((2, _BM, _C), lambda i: (0, i, 0)),
            pl.BlockSpec((_BM, _C), lambda i: (i, 0)),
            pl.BlockSpec((_C, cout), lambda i: (0, 0)),
            pl.BlockSpec((_C, cout), lambda i: (0, 0)),
            pl.BlockSpec((1, cout), lambda i: (0, 0)),
        ],
        out_specs=out_specs,
        out_shape=out_shape,
    )


_layer1 = _make_layer(128, relu=True, split_out=False)
_layer2 = _make_layer(256, relu=True, split_out=True)


def _l3_body(pa_ref, pb_ref, ha_ref, hb_ref, wa_ref, wb_ref,
             wra_ref, wrb_ref, b_ref, bat_ref, o_ref, acc_ref, cnt_ref):
    i = pl.program_id(0)

    @pl.when(i == 0)
    def _():
        acc_ref[...] = jnp.zeros_like(acc_ref)
        cnt_ref[...] = jnp.zeros_like(cnt_ref)

    h3 = jnp.dot(pa_ref[0] + pa_ref[1], wa_ref[...],
                 preferred_element_type=jnp.float32)
    h3 = h3 + jnp.dot(pb_ref[0] + pb_ref[1], wb_ref[...],
                      preferred_element_type=jnp.float32)
    h3 = h3 + jnp.dot(ha_ref[...], wra_ref[...],
                      preferred_element_type=jnp.float32)
    h3 = h3 + jnp.dot(hb_ref[...], wrb_ref[...],
                      preferred_element_type=jnp.float32)
    h3 = h3 + b_ref[...]
    bb = bat_ref[0]
    gids = lax.broadcasted_iota(jnp.int32, (_G, _BM), 0)
    onehot = jnp.where(bb == gids, 1.0, 0.0).astype(jnp.float32)
    acc_ref[...] += jnp.dot(onehot, h3, preferred_element_type=jnp.float32)
    cnt_ref[...] += jnp.broadcast_to(
        jnp.sum(onehot, axis=1, keepdims=True), cnt_ref.shape)

    @pl.when(i == pl.num_programs(0) - 1)
    def _():
        o_ref[...] = acc_ref[...] / jnp.maximum(cnt_ref[:, 0:1], 1.0)


_l3 = pl.pallas_call(
    _l3_body,
    grid=(_N // _BM,),
    in_specs=[
        pl.BlockSpec((2, _BM, _C), lambda i: (0, i, 0)),
        pl.BlockSpec((2, _BM, _C), lambda i: (0, i, 0)),
        pl.BlockSpec((_BM, _C), lambda i: (i, 0)),
        pl.BlockSpec((_BM, _C), lambda i: (i, 0)),
        pl.BlockSpec((_C, 512), lambda i: (0, 0)),
        pl.BlockSpec((_C, 512), lambda i: (0, 0)),
        pl.BlockSpec((_C, 512), lambda i: (0, 0)),
        pl.BlockSpec((_C, 512), lambda i: (0, 0)),
        pl.BlockSpec((1, 512), lambda i: (0, 0)),
        pl.BlockSpec((1, 1, _BM), lambda i: (i, 0, 0)),
    ],
    out_specs=pl.BlockSpec((_G, 512), lambda i: (0, 0)),
    out_shape=jax.ShapeDtypeStruct((_G, 512), jnp.float32),
    scratch_shapes=[
        pltpu.VMEM((_G, 512), jnp.float32),
        pltpu.VMEM((_G, 128), jnp.float32),
    ],
)


def kernel(x, edge_index, batch, W1_rel, W1_root, b1,
           W2_rel, W2_root, b2, W3_rel, W3_root, b3):
    src = edge_index[0].reshape(_NW, _NCH, _B)
    dst = edge_index[1].reshape(_NW, _NCH, _B)
    zeros = jnp.zeros((_N, _C), jnp.float32)
    _agg = _get_agg()

    p1 = _agg(x, src, dst, zeros)
    h1 = _layer1(p1, x, W1_rel, W1_root, b1.reshape(1, -1))
    p2 = _agg(h1, src, dst, zeros)
    h2a, h2b = _layer2(p2, h1, W2_rel, W2_root, b2.reshape(1, -1))
    pa = _agg(h2a, src, dst, zeros)
    pb = _agg(h2b, src, dst, zeros)
    out = _l3(pa, pb, h2a, h2b,
              W3_rel[:_C], W3_rel[_C:], W3_root[:_C], W3_root[_C:],
              b3.reshape(1, -1), batch.reshape(_N // _BM, 1, _BM))
    return out


# speedup vs baseline: 1.0131x; 1.0131x over previous
import functools

import jax
import jax.numpy as jnp
from jax import lax
from jax.experimental import pallas as pl
from jax.experimental.pallas import tpu as pltpu
from jax.experimental.pallas import tpu_sc as plsc

_N = 10000
_E = 320000
_G = 16
_C = 128
_NC, _NS = 2, 16
_NW = _NC * _NS
_EW = _E // _NW
_B = 80
_NCH = _EW // _B
_NSTG = 5
_WIN = _NCH // _NSTG
_RPS = 624
_TAIL = _N - _RPS * _NS
_BM = 2000


def _agg_body(h_hbm, src_hbm, dst_hbm, z_hbm, out_hbm,
              src_v, dst_v, r0, r1, table_sh, g0, g1):
    c = lax.axis_index("c")
    s = lax.axis_index("s")
    w = c * _NS + s
    pltpu.sync_copy(z_hbm.at[pl.ds(s * _RPS, _RPS)],
                    table_sh.at[pl.ds(s * _RPS, _RPS)])

    @pl.when(s == _NS - 1)
    def _():
        pltpu.sync_copy(z_hbm.at[pl.ds(_RPS * _NS, _TAIL)],
                        table_sh.at[pl.ds(_RPS * _NS, _TAIL)])
    plsc.subcore_barrier()

    def _start(t, buf, sem):
        pltpu.async_copy(h_hbm.at[src_v.at[t]], buf, sem)

    def _finish(buf, sem, t):
        pltpu.make_async_copy(h_hbm.at[src_v.at[0]], buf, sem).wait()
        pltpu.sync_copy(buf, table_sh.at[dst_v.at[t]], add=True)

    for stg in range(_NSTG):
        pltpu.sync_copy(src_hbm.at[w, stg], src_v)
        pltpu.sync_copy(dst_hbm.at[w, stg], dst_v)
        _start(0, r0, g0)

        def pair(j, carry):
            t0 = 2 * j
            _start(t0 + 1, r1, g1)
            _finish(r0, g0, t0)
            _start(t0 + 2, r0, g0)
            _finish(r1, g1, t0 + 1)
            return carry

        lax.fori_loop(0, (_WIN - 1) // 2, pair, 0)
        _finish(r0, g0, _WIN - 1)
    plsc.subcore_barrier()
    pltpu.sync_copy(table_sh.at[pl.ds(s * _RPS, _RPS)],
                    out_hbm.at[c, pl.ds(s * _RPS, _RPS)])

    @pl.when(s == _NS - 1)
    def _():
        pltpu.sync_copy(table_sh.at[pl.ds(_RPS * _NS, _TAIL)],
                        out_hbm.at[c, pl.ds(_RPS * _NS, _TAIL)])


@functools.cache
def _get_agg():
    return pl.kernel(
        _agg_body,
        out_type=jax.ShapeDtypeStruct((_NC, _N, _C), jnp.float32),
        mesh=plsc.VectorSubcoreMesh(core_axis_name="c", subcore_axis_name="s"),
        scratch_types=(
            [pltpu.VMEM((_WIN, _B), jnp.int32)] * 2
            + [pltpu.VMEM((_B, _C), jnp.float32)] * 2
            + [pltpu.VMEM_SHARED((_N, _C), jnp.float32)]
            + [pltpu.SemaphoreType.DMA] * 2
        ),
    )


def _layer_body(relu, split_out, p_ref, x_ref, wrel_ref, wroot_ref, b_ref,
                *o_refs):
    agg = p_ref[0] + p_ref[1]
    acc = jnp.dot(agg, wrel_ref[...], preferred_element_type=jnp.float32)
    acc = acc + jnp.dot(x_ref[...], wroot_ref[...],
                        preferred_element_type=jnp.float32)
    acc = acc + b_ref[...]
    if relu:
        acc = jnp.maximum(acc, 0.0)
    if split_out:
        o_refs[0][...] = acc[:, :_C]
        o_refs[1][...] = acc[:, _C:]
    else:
        o_refs[0][...] = acc


def _make_layer(cout, relu, split_out):
    out_shape = (
        [jax.ShapeDtypeStruct((_N, _C), jnp.float32)] * 2 if split_out
        else jax.ShapeDtypeStruct((_N, cout), jnp.float32))
    out_specs = (
        [pl.BlockSpec((_BM, _C), lambda i: (i, 0))] * 2 if split_out
        else pl.BlockSpec((_BM, cout), lambda i: (i, 0)))
    return pl.pallas_call(
        functools.partial(_layer_body, relu, split_out),
        grid=(_N // _BM,),
        in_specs=[
            pl.BlockSpec((2, _BM, _C), lambda i: (0, i, 0)),
            pl.BlockSpec((_BM, _C), lambda i: (i, 0)),
            pl.BlockSpec((_C, cout), lambda i: (0, 0)),
            pl.BlockSpec((_C, cout), lambda i: (0, 0)),
            pl.BlockSpec((1, cout), lambda i: (0, 0)),
        ],
        out_specs=out_specs,
        out_shape=out_shape,
    )


_layer1 = _make_layer(128, relu=True, split_out=False)
_layer2 = _make_layer(256, relu=True, split_out=True)


def _l3_body(pa_ref, pb_ref, ha_ref, hb_ref, wa_ref, wb_ref,
             wra_ref, wrb_ref, b_ref, bat_ref, o_ref, acc_ref, cnt_ref):
    i = pl.program_id(0)

    @pl.when(i == 0)
    def _():
        acc_ref[...] = jnp.zeros_like(acc_ref)
        cnt_ref[...] = jnp.zeros_like(cnt_ref)

    h3 = jnp.dot(pa_ref[0] + pa_ref[1], wa_ref[...],
                 preferred_element_type=jnp.float32)
    h3 = h3 + jnp.dot(pb_ref[0] + pb_ref[1], wb_ref[...],
                      preferred_element_type=jnp.float32)
    h3 = h3 + jnp.dot(ha_ref[...], wra_ref[...],
                      preferred_element_type=jnp.float32)
    h3 = h3 + jnp.dot(hb_ref[...], wrb_ref[...],
                      preferred_element_type=jnp.float32)
    h3 = h3 + b_ref[...]
    bb = bat_ref[0]
    gids = lax.broadcasted_iota(jnp.int32, (_G, _BM), 0)
    onehot = jnp.where(bb == gids, 1.0, 0.0).astype(jnp.float32)
    acc_ref[...] += jnp.dot(onehot, h3, preferred_element_type=jnp.float32)
    cnt_ref[...] += jnp.broadcast_to(
        jnp.sum(onehot, axis=1, keepdims=True), cnt_ref.shape)

    @pl.when(i == pl.num_programs(0) - 1)
    def _():
        o_ref[...] = acc_ref[...] / jnp.maximum(cnt_ref[:, 0:1], 1.0)


_l3 = pl.pallas_call(
    _l3_body,
    grid=(_N // _BM,),
    in_specs=[
        pl.BlockSpec((2, _BM, _C), lambda i: (0, i, 0)),
        pl.BlockSpec((2, _BM, _C), lambda i: (0, i, 0)),
        pl.BlockSpec((_BM, _C), lambda i: (i, 0)),
        pl.BlockSpec((_BM, _C), lambda i: (i, 0)),
        pl.BlockSpec((_C, 512), lambda i: (0, 0)),
        pl.BlockSpec((_C, 512), lambda i: (0, 0)),
        pl.BlockSpec((_C, 512), lambda i: (0, 0)),
        pl.BlockSpec((_C, 512), lambda i: (0, 0)),
        pl.BlockSpec((1, 512), lambda i: (0, 0)),
        pl.BlockSpec((1, 1, _BM), lambda i: (i, 0, 0)),
    ],
    out_specs=pl.BlockSpec((_G, 512), lambda i: (0, 0)),
    out_shape=jax.ShapeDtypeStruct((_G, 512), jnp.float32),
    scratch_shapes=[
        pltpu.VMEM((_G, 512), jnp.float32),
        pltpu.VMEM((_G, 128), jnp.float32),
    ],
)


def kernel(x, edge_index, batch, W1_rel, W1_root, b1,
           W2_rel, W2_root, b2, W3_rel, W3_root, b3):
    src = edge_index[0].reshape(_NW, _NSTG, _WIN, _B)
    dst = edge_index[1].reshape(_NW, _NSTG, _WIN, _B)
    zeros = jnp.zeros((_N, _C), jnp.float32)
    _agg = _get_agg()

    p1 = _agg(x, src, dst, zeros)
    h1 = _layer1(p1, x, W1_rel, W1_root, b1.reshape(1, -1))
    p2 = _agg(h1, src, dst, zeros)
    h2a, h2b = _layer2(p2, h1, W2_rel, W2_root, b2.reshape(1, -1))
    pa = _agg(h2a, src, dst, zeros)
    pb = _agg(h2b, src, dst, zeros)
    out = _l3(pa, pb, h2a, h2b,
              W3_rel[:_C], W3_rel[_C:], W3_root[:_C], W3_root[_C:],
              b3.reshape(1, -1), batch.reshape(_N // _BM, 1, _BM))
    return out


# speedup vs baseline: 1.0792x; 1.0653x over previous
import functools

import jax
import jax.numpy as jnp
from jax import lax
from jax.experimental import pallas as pl
from jax.experimental.pallas import tpu as pltpu
from jax.experimental.pallas import tpu_sc as plsc

_N = 10000
_E = 320000
_G = 16
_C = 128
_NC, _NS = 2, 16
_NW = _NC * _NS
_EW = _E // _NW
_B = 100
_NCH = _EW // _B
_NSTG = 4
_WIN = _NCH // _NSTG
_RPS = 624
_TAIL = _N - _RPS * _NS
_BM = 2000


def _agg_body(h_hbm, src_hbm, dst_hbm, z_hbm, out_hbm,
              src_v, dst_v, r0, r1, table_sh, g0, g1):
    c = lax.axis_index("c")
    s = lax.axis_index("s")
    w = c * _NS + s
    pltpu.sync_copy(z_hbm.at[pl.ds(s * _RPS, _RPS)],
                    table_sh.at[pl.ds(s * _RPS, _RPS)])

    @pl.when(s == _NS - 1)
    def _():
        pltpu.sync_copy(z_hbm.at[pl.ds(_RPS * _NS, _TAIL)],
                        table_sh.at[pl.ds(_RPS * _NS, _TAIL)])
    plsc.subcore_barrier()

    def _start(t, buf, sem):
        pltpu.async_copy(h_hbm.at[src_v.at[t]], buf, sem)

    def _finish(buf, sem, t):
        pltpu.make_async_copy(h_hbm.at[src_v.at[0]], buf, sem).wait()
        pltpu.sync_copy(buf, table_sh.at[dst_v.at[t]], add=True)

    for stg in range(_NSTG):
        pltpu.sync_copy(src_hbm.at[w, stg], src_v)
        pltpu.sync_copy(dst_hbm.at[w, stg], dst_v)
        _start(0, r0, g0)

        def pair(j, carry):
            t0 = 2 * j
            _start(t0 + 1, r1, g1)
            _finish(r0, g0, t0)
            _start(t0 + 2, r0, g0)
            _finish(r1, g1, t0 + 1)
            return carry

        lax.fori_loop(0, (_WIN - 1) // 2, pair, 0)
        _finish(r0, g0, _WIN - 1)
    plsc.subcore_barrier()
    pltpu.sync_copy(table_sh.at[pl.ds(s * _RPS, _RPS)],
                    out_hbm.at[c, pl.ds(s * _RPS, _RPS)])

    @pl.when(s == _NS - 1)
    def _():
        pltpu.sync_copy(table_sh.at[pl.ds(_RPS * _NS, _TAIL)],
                        out_hbm.at[c, pl.ds(_RPS * _NS, _TAIL)])


@functools.cache
def _get_agg():
    return pl.kernel(
        _agg_body,
        out_type=jax.ShapeDtypeStruct((_NC, _N, _C), jnp.float32),
        mesh=plsc.VectorSubcoreMesh(core_axis_name="c", subcore_axis_name="s"),
        scratch_types=(
            [pltpu.VMEM((_WIN, _B), jnp.int32)] * 2
            + [pltpu.VMEM((_B, _C), jnp.float32)] * 2
            + [pltpu.VMEM_SHARED((_N, _C), jnp.float32)]
            + [pltpu.SemaphoreType.DMA] * 2
        ),
    )


def _layer_body(relu, split_out, p_ref, x_ref, wrel_ref, wroot_ref, b_ref,
                *o_refs):
    agg = p_ref[0] + p_ref[1]
    acc = jnp.dot(agg, wrel_ref[...], preferred_element_type=jnp.float32)
    acc = acc + jnp.dot(x_ref[...], wroot_ref[...],
                        preferred_element_type=jnp.float32)
    acc = acc + b_ref[...]
    if relu:
        acc = jnp.maximum(acc, 0.0)
    if split_out:
        o_refs[0][...] = acc[:, :_C]
        o_refs[1][...] = acc[:, _C:]
    else:
        o_refs[0][...] = acc


def _make_layer(cout, relu, split_out):
    out_shape = (
        [jax.ShapeDtypeStruct((_N, _C), jnp.float32)] * 2 if split_out
        else jax.ShapeDtypeStruct((_N, cout), jnp.float32))
    out_specs = (
        [pl.BlockSpec((_BM, _C), lambda i: (i, 0))] * 2 if split_out
        else pl.BlockSpec((_BM, cout), lambda i: (i, 0)))
    return pl.pallas_call(
        functools.partial(_layer_body, relu, split_out),
        grid=(_N // _BM,),
        in_specs=[
            pl.BlockSpec((2, _BM, _C), lambda i: (0, i, 0)),
            pl.BlockSpec((_BM, _C), lambda i: (i, 0)),
            pl.BlockSpec((_C, cout), lambda i: (0, 0)),
            pl.BlockSpec((_C, cout), lambda i: (0, 0)),
            pl.BlockSpec((1, cout), lambda i: (0, 0)),
        ],
        out_specs=out_specs,
        out_shape=out_shape,
    )


_layer1 = _make_layer(128, relu=True, split_out=False)
_layer2 = _make_layer(256, relu=True, split_out=True)


def _l3_body(pa_ref, pb_ref, ha_ref, hb_ref, wa_ref, wb_ref,
             wra_ref, wrb_ref, b_ref, bat_ref, o_ref, acc_ref, cnt_ref):
    i = pl.program_id(0)

    @pl.when(i == 0)
    def _():
        acc_ref[...] = jnp.zeros_like(acc_ref)
        cnt_ref[...] = jnp.zeros_like(cnt_ref)

    h3 = jnp.dot(pa_ref[0] + pa_ref[1], wa_ref[...],
                 preferred_element_type=jnp.float32)
    h3 = h3 + jnp.dot(pb_ref[0] + pb_ref[1], wb_ref[...],
                      preferred_element_type=jnp.float32)
    h3 = h3 + jnp.dot(ha_ref[...], wra_ref[...],
                      preferred_element_type=jnp.float32)
    h3 = h3 + jnp.dot(hb_ref[...], wrb_ref[...],
                      preferred_element_type=jnp.float32)
    h3 = h3 + b_ref[...]
    bb = bat_ref[0]
    gids = lax.broadcasted_iota(jnp.int32, (_G, _BM), 0)
    onehot = jnp.where(bb == gids, 1.0, 0.0).astype(jnp.float32)
    acc_ref[...] += jnp.dot(onehot, h3, preferred_element_type=jnp.float32)
    cnt_ref[...] += jnp.broadcast_to(
        jnp.sum(onehot, axis=1, keepdims=True), cnt_ref.shape)

    @pl.when(i == pl.num_programs(0) - 1)
    def _():
        o_ref[...] = acc_ref[...] / jnp.maximum(cnt_ref[:, 0:1], 1.0)


_l3 = pl.pallas_call(
    _l3_body,
    grid=(_N // _BM,),
    in_specs=[
        pl.BlockSpec((2, _BM, _C), lambda i: (0, i, 0)),
        pl.BlockSpec((2, _BM, _C), lambda i: (0, i, 0)),
        pl.BlockSpec((_BM, _C), lambda i: (i, 0)),
        pl.BlockSpec((_BM, _C), lambda i: (i, 0)),
        pl.BlockSpec((_C, 512), lambda i: (0, 0)),
        pl.BlockSpec((_C, 512), lambda i: (0, 0)),
        pl.BlockSpec((_C, 512), lambda i: (0, 0)),
        pl.BlockSpec((_C, 512), lambda i: (0, 0)),
        pl.BlockSpec((1, 512), lambda i: (0, 0)),
        pl.BlockSpec((1, 1, _BM), lambda i: (i, 0, 0)),
    ],
    out_specs=pl.BlockSpec((_G, 512), lambda i: (0, 0)),
    out_shape=jax.ShapeDtypeStruct((_G, 512), jnp.float32),
    scratch_shapes=[
        pltpu.VMEM((_G, 512), jnp.float32),
        pltpu.VMEM((_G, 128), jnp.float32),
    ],
)


def kernel(x, edge_index, batch, W1_rel, W1_root, b1,
           W2_rel, W2_root, b2, W3_rel, W3_root, b3):
    src = edge_index[0].reshape(_NW, _NSTG, _WIN, _B)
    dst = edge_index[1].reshape(_NW, _NSTG, _WIN, _B)
    zeros = jnp.zeros((_N, _C), jnp.float32)
    _agg = _get_agg()

    p1 = _agg(x, src, dst, zeros)
    h1 = _layer1(p1, x, W1_rel, W1_root, b1.reshape(1, -1))
    p2 = _agg(h1, src, dst, zeros)
    h2a, h2b = _layer2(p2, h1, W2_rel, W2_root, b2.reshape(1, -1))
    pa = _agg(h2a, src, dst, zeros)
    pb = _agg(h2b, src, dst, zeros)
    out = _l3(pa, pb, h2a, h2b,
              W3_rel[:_C], W3_rel[_C:], W3_root[:_C], W3_root[_C:],
              b3.reshape(1, -1), batch.reshape(_N // _BM, 1, _BM))
    return out


# speedup vs baseline: 1.1043x; 1.0232x over previous
import functools

import jax
import jax.numpy as jnp
from jax import lax
from jax.experimental import pallas as pl
from jax.experimental.pallas import tpu as pltpu
from jax.experimental.pallas import tpu_sc as plsc

_N = 10000
_E = 320000
_G = 16
_C = 128
_NC, _NS = 2, 16
_NW = _NC * _NS
_EW = _E // _NW
_B = 125
_NCH = _EW // _B
_NSTG = 4
_WIN = _NCH // _NSTG
_RPS = 624
_TAIL = _N - _RPS * _NS
_BM = 2000


def _agg_body(h_hbm, src_hbm, dst_hbm, z_hbm, out_hbm,
              src_v, dst_v, r0, r1, table_sh, g0, g1):
    c = lax.axis_index("c")
    s = lax.axis_index("s")
    w = c * _NS + s
    pltpu.sync_copy(z_hbm.at[pl.ds(s * _RPS, _RPS)],
                    table_sh.at[pl.ds(s * _RPS, _RPS)])

    @pl.when(s == _NS - 1)
    def _():
        pltpu.sync_copy(z_hbm.at[pl.ds(_RPS * _NS, _TAIL)],
                        table_sh.at[pl.ds(_RPS * _NS, _TAIL)])
    plsc.subcore_barrier()

    def _start(t, buf, sem):
        pltpu.async_copy(h_hbm.at[src_v.at[t]], buf, sem)

    def _finish(buf, sem, t):
        pltpu.make_async_copy(h_hbm.at[src_v.at[0]], buf, sem).wait()
        pltpu.sync_copy(buf, table_sh.at[dst_v.at[t]], add=True)

    for stg in range(_NSTG):
        pltpu.sync_copy(src_hbm.at[w, stg], src_v)
        pltpu.sync_copy(dst_hbm.at[w, stg], dst_v)
        _start(0, r0, g0)

        def pair(j, carry):
            t0 = 2 * j
            _start(t0 + 1, r1, g1)
            _finish(r0, g0, t0)
            _start(t0 + 2, r0, g0)
            _finish(r1, g1, t0 + 1)
            return carry

        lax.fori_loop(0, (_WIN - 1) // 2 if _WIN % 2 else (_WIN - 2) // 2,
                      pair, 0)
        if _WIN % 2:
            _finish(r0, g0, _WIN - 1)
        else:
            _start(_WIN - 1, r1, g1)
            _finish(r0, g0, _WIN - 2)
            _finish(r1, g1, _WIN - 1)
    plsc.subcore_barrier()
    pltpu.sync_copy(table_sh.at[pl.ds(s * _RPS, _RPS)],
                    out_hbm.at[c, pl.ds(s * _RPS, _RPS)])

    @pl.when(s == _NS - 1)
    def _():
        pltpu.sync_copy(table_sh.at[pl.ds(_RPS * _NS, _TAIL)],
                        out_hbm.at[c, pl.ds(_RPS * _NS, _TAIL)])


@functools.cache
def _get_agg():
    return pl.kernel(
        _agg_body,
        out_type=jax.ShapeDtypeStruct((_NC, _N, _C), jnp.float32),
        mesh=plsc.VectorSubcoreMesh(core_axis_name="c", subcore_axis_name="s"),
        scratch_types=(
            [pltpu.VMEM((_WIN, _B), jnp.int32)] * 2
            + [pltpu.VMEM((_B, _C), jnp.float32)] * 2
            + [pltpu.VMEM_SHARED((_N, _C), jnp.float32)]
            + [pltpu.SemaphoreType.DMA] * 2
        ),
    )


def _layer_body(relu, split_out, p_ref, x_ref, wrel_ref, wroot_ref, b_ref,
                *o_refs):
    agg = p_ref[0] + p_ref[1]
    acc = jnp.dot(agg, wrel_ref[...], preferred_element_type=jnp.float32)
    acc = acc + jnp.dot(x_ref[...], wroot_ref[...],
                        preferred_element_type=jnp.float32)
    acc = acc + b_ref[...]
    if relu:
        acc = jnp.maximum(acc, 0.0)
    if split_out:
        o_refs[0][...] = acc[:, :_C]
        o_refs[1][...] = acc[:, _C:]
    else:
        o_refs[0][...] = acc


def _make_layer(cout, relu, split_out):
    out_shape = (
        [jax.ShapeDtypeStruct((_N, _C), jnp.float32)] * 2 if split_out
        else jax.ShapeDtypeStruct((_N, cout), jnp.float32))
    out_specs = (
        [pl.BlockSpec((_BM, _C), lambda i: (i, 0))] * 2 if split_out
        else pl.BlockSpec((_BM, cout), lambda i: (i, 0)))
    return pl.pallas_call(
        functools.partial(_layer_body, relu, split_out),
        grid=(_N // _BM,),
        in_specs=[
            pl.BlockSpec((2, _BM, _C), lambda i: (0, i, 0)),
            pl.BlockSpec((_BM, _C), lambda i: (i, 0)),
            pl.BlockSpec((_C, cout), lambda i: (0, 0)),
            pl.BlockSpec((_C, cout), lambda i: (0, 0)),
            pl.BlockSpec((1, cout), lambda i: (0, 0)),
        ],
        out_specs=out_specs,
        out_shape=out_shape,
    )


_layer1 = _make_layer(128, relu=True, split_out=False)
_layer2 = _make_layer(256, relu=True, split_out=True)


def _l3_body(pa_ref, pb_ref, ha_ref, hb_ref, wa_ref, wb_ref,
             wra_ref, wrb_ref, b_ref, bat_ref, o_ref, acc_ref, cnt_ref):
    i = pl.program_id(0)

    @pl.when(i == 0)
    def _():
        acc_ref[...] = jnp.zeros_like(acc_ref)
        cnt_ref[...] = jnp.zeros_like(cnt_ref)

    h3 = jnp.dot(pa_ref[0] + pa_ref[1], wa_ref[...],
                 preferred_element_type=jnp.float32)
    h3 = h3 + jnp.dot(pb_ref[0] + pb_ref[1], wb_ref[...],
                      preferred_element_type=jnp.float32)
    h3 = h3 + jnp.dot(ha_ref[...], wra_ref[...],
                      preferred_element_type=jnp.float32)
    h3 = h3 + jnp.dot(hb_ref[...], wrb_ref[...],
                      preferred_element_type=jnp.float32)
    h3 = h3 + b_ref[...]
    bb = bat_ref[0]
    gids = lax.broadcasted_iota(jnp.int32, (_G, _BM), 0)
    onehot = jnp.where(bb == gids, 1.0, 0.0).astype(jnp.float32)
    acc_ref[...] += jnp.dot(onehot, h3, preferred_element_type=jnp.float32)
    cnt_ref[...] += jnp.broadcast_to(
        jnp.sum(onehot, axis=1, keepdims=True), cnt_ref.shape)

    @pl.when(i == pl.num_programs(0) - 1)
    def _():
        o_ref[...] = acc_ref[...] / jnp.maximum(cnt_ref[:, 0:1], 1.0)


_l3 = pl.pallas_call(
    _l3_body,
    grid=(_N // _BM,),
    in_specs=[
        pl.BlockSpec((2, _BM, _C), lambda i: (0, i, 0)),
        pl.BlockSpec((2, _BM, _C), lambda i: (0, i, 0)),
        pl.BlockSpec((_BM, _C), lambda i: (i, 0)),
        pl.BlockSpec((_BM, _C), lambda i: (i, 0)),
        pl.BlockSpec((_C, 512), lambda i: (0, 0)),
        pl.BlockSpec((_C, 512), lambda i: (0, 0)),
        pl.BlockSpec((_C, 512), lambda i: (0, 0)),
        pl.BlockSpec((_C, 512), lambda i: (0, 0)),
        pl.BlockSpec((1, 512), lambda i: (0, 0)),
        pl.BlockSpec((1, 1, _BM), lambda i: (i, 0, 0)),
    ],
    out_specs=pl.BlockSpec((_G, 512), lambda i: (0, 0)),
    out_shape=jax.ShapeDtypeStruct((_G, 512), jnp.float32),
    scratch_shapes=[
        pltpu.VMEM((_G, 512), jnp.float32),
        pltpu.VMEM((_G, 128), jnp.float32),
    ],
)


def kernel(x, edge_index, batch, W1_rel, W1_root, b1,
           W2_rel, W2_root, b2, W3_rel, W3_root, b3):
    src = edge_index[0].reshape(_NW, _NSTG, _WIN, _B)
    dst = edge_index[1].reshape(_NW, _NSTG, _WIN, _B)
    zeros = jnp.zeros((_N, _C), jnp.float32)
    _agg = _get_agg()

    p1 = _agg(x, src, dst, zeros)
    h1 = _layer1(p1, x, W1_rel, W1_root, b1.reshape(1, -1))
    p2 = _agg(h1, src, dst, zeros)
    h2a, h2b = _layer2(p2, h1, W2_rel, W2_root, b2.reshape(1, -1))
    pa = _agg(h2a, src, dst, zeros)
    pb = _agg(h2b, src, dst, zeros)
    out = _l3(pa, pb, h2a, h2b,
              W3_rel[:_C], W3_rel[_C:], W3_root[:_C], W3_root[_C:],
              b3.reshape(1, -1), batch.reshape(_N // _BM, 1, _BM))
    return out


# speedup vs baseline: 1.1187x; 1.0130x over previous
import functools

import jax
import jax.numpy as jnp
from jax import lax
from jax.experimental import pallas as pl
from jax.experimental.pallas import tpu as pltpu
from jax.experimental.pallas import tpu_sc as plsc

_N = 10000
_E = 320000
_G = 16
_C = 128
_NC, _NS = 2, 16
_NW = _NC * _NS
_EW = _E // _NW
_B = 125
_NCH = _EW // _B
_NSTG = 4
_WIN = _NCH // _NSTG
_RPS = 624
_TAIL = _N - _RPS * _NS
_BM = 2000


def _agg_body(h_hbm, src_hbm, dst_hbm, z_hbm, out_hbm,
              src_v, dst_v, r0, r1, table_sh, g0, g1):
    c = lax.axis_index("c")
    s = lax.axis_index("s")
    w = c * _NS + s

    def _start(t, buf, sem):
        pltpu.async_copy(h_hbm.at[src_v.at[t]], buf, sem)

    def _finish(buf, sem, t):
        pltpu.make_async_copy(h_hbm.at[src_v.at[0]], buf, sem).wait()
        pltpu.sync_copy(buf, table_sh.at[dst_v.at[t]], add=True)

    for stg in range(_NSTG):
        pltpu.sync_copy(src_hbm.at[w, stg], src_v)
        pltpu.sync_copy(dst_hbm.at[w, stg], dst_v)
        _start(0, r0, g0)

        if stg == 0:
            pltpu.sync_copy(z_hbm.at[pl.ds(s * _RPS, _RPS)],
                            table_sh.at[pl.ds(s * _RPS, _RPS)])

            @pl.when(s == _NS - 1)
            def _():
                pltpu.sync_copy(z_hbm.at[pl.ds(_RPS * _NS, _TAIL)],
                                table_sh.at[pl.ds(_RPS * _NS, _TAIL)])
            plsc.subcore_barrier()

        def pair(j, carry):
            t0 = 2 * j
            _start(t0 + 1, r1, g1)
            _finish(r0, g0, t0)
            _start(t0 + 2, r0, g0)
            _finish(r1, g1, t0 + 1)
            return carry

        lax.fori_loop(0, (_WIN - 1) // 2 if _WIN % 2 else (_WIN - 2) // 2,
                      pair, 0)
        if _WIN % 2:
            _finish(r0, g0, _WIN - 1)
        else:
            _start(_WIN - 1, r1, g1)
            _finish(r0, g0, _WIN - 2)
            _finish(r1, g1, _WIN - 1)
    plsc.subcore_barrier()
    pltpu.sync_copy(table_sh.at[pl.ds(s * _RPS, _RPS)],
                    out_hbm.at[c, pl.ds(s * _RPS, _RPS)])

    @pl.when(s == _NS - 1)
    def _():
        pltpu.sync_copy(table_sh.at[pl.ds(_RPS * _NS, _TAIL)],
                        out_hbm.at[c, pl.ds(_RPS * _NS, _TAIL)])


@functools.cache
def _get_agg():
    return pl.kernel(
        _agg_body,
        out_type=jax.ShapeDtypeStruct((_NC, _N, _C), jnp.float32),
        mesh=plsc.VectorSubcoreMesh(core_axis_name="c", subcore_axis_name="s"),
        scratch_types=(
            [pltpu.VMEM((_WIN, _B), jnp.int32)] * 2
            + [pltpu.VMEM((_B, _C), jnp.float32)] * 2
            + [pltpu.VMEM_SHARED((_N, _C), jnp.float32)]
            + [pltpu.SemaphoreType.DMA] * 2
        ),
    )


def _layer_body(relu, split_out, p_ref, x_ref, wrel_ref, wroot_ref, b_ref,
                *o_refs):
    agg = p_ref[0] + p_ref[1]
    acc = jnp.dot(agg, wrel_ref[...], preferred_element_type=jnp.float32)
    acc = acc + jnp.dot(x_ref[...], wroot_ref[...],
                        preferred_element_type=jnp.float32)
    acc = acc + b_ref[...]
    if relu:
        acc = jnp.maximum(acc, 0.0)
    if split_out:
        o_refs[0][...] = acc[:, :_C]
        o_refs[1][...] = acc[:, _C:]
    else:
        o_refs[0][...] = acc


def _make_layer(cout, relu, split_out):
    out_shape = (
        [jax.ShapeDtypeStruct((_N, _C), jnp.float32)] * 2 if split_out
        else jax.ShapeDtypeStruct((_N, cout), jnp.float32))
    out_specs = (
        [pl.BlockSpec((_BM, _C), lambda i: (i, 0))] * 2 if split_out
        else pl.BlockSpec((_BM, cout), lambda i: (i, 0)))
    return pl.pallas_call(
        functools.partial(_layer_body, relu, split_out),
        grid=(_N // _BM,),
        in_specs=[
            pl.BlockSpec((2, _BM, _C), lambda i: (0, i, 0)),
            pl.BlockSpec((_BM, _C), lambda i: (i, 0)),
            pl.BlockSpec((_C, cout), lambda i: (0, 0)),
            pl.BlockSpec((_C, cout), lambda i: (0, 0)),
            pl.BlockSpec((1, cout), lambda i: (0, 0)),
        ],
        out_specs=out_specs,
        out_shape=out_shape,
    )


_layer1 = _make_layer(128, relu=True, split_out=False)
_layer2 = _make_layer(256, relu=True, split_out=True)


def _l3_body(pa_ref, pb_ref, ha_ref, hb_ref, wa_ref, wb_ref,
             wra_ref, wrb_ref, b_ref, bat_ref, o_ref, acc_ref, cnt_ref):
    i = pl.program_id(0)

    @pl.when(i == 0)
    def _():
        acc_ref[...] = jnp.zeros_like(acc_ref)
        cnt_ref[...] = jnp.zeros_like(cnt_ref)

    h3 = jnp.dot(pa_ref[0] + pa_ref[1], wa_ref[...],
                 preferred_element_type=jnp.float32)
    h3 = h3 + jnp.dot(pb_ref[0] + pb_ref[1], wb_ref[...],
                      preferred_element_type=jnp.float32)
    h3 = h3 + jnp.dot(ha_ref[...], wra_ref[...],
                      preferred_element_type=jnp.float32)
    h3 = h3 + jnp.dot(hb_ref[...], wrb_ref[...],
                      preferred_element_type=jnp.float32)
    h3 = h3 + b_ref[...]
    bb = bat_ref[0]
    gids = lax.broadcasted_iota(jnp.int32, (_G, _BM), 0)
    onehot = jnp.where(bb == gids, 1.0, 0.0).astype(jnp.float32)
    acc_ref[...] += jnp.dot(onehot, h3, preferred_element_type=jnp.float32)
    cnt_ref[...] += jnp.broadcast_to(
        jnp.sum(onehot, axis=1, keepdims=True), cnt_ref.shape)

    @pl.when(i == pl.num_programs(0) - 1)
    def _():
        o_ref[...] = acc_ref[...] / jnp.maximum(cnt_ref[:, 0:1], 1.0)


_l3 = pl.pallas_call(
    _l3_body,
    grid=(_N // _BM,),
    in_specs=[
        pl.BlockSpec((2, _BM, _C), lambda i: (0, i, 0)),
        pl.BlockSpec((2, _BM, _C), lambda i: (0, i, 0)),
        pl.BlockSpec((_BM, _C), lambda i: (i, 0)),
        pl.BlockSpec((_BM, _C), lambda i: (i, 0)),
        pl.BlockSpec((_C, 512), lambda i: (0, 0)),
        pl.BlockSpec((_C, 512), lambda i: (0, 0)),
        pl.BlockSpec((_C, 512), lambda i: (0, 0)),
        pl.BlockSpec((_C, 512), lambda i: (0, 0)),
        pl.BlockSpec((1, 512), lambda i: (0, 0)),
        pl.BlockSpec((1, 1, _BM), lambda i: (i, 0, 0)),
    ],
    out_specs=pl.BlockSpec((_G, 512), lambda i: (0, 0)),
    out_shape=jax.ShapeDtypeStruct((_G, 512), jnp.float32),
    scratch_shapes=[
        pltpu.VMEM((_G, 512), jnp.float32),
        pltpu.VMEM((_G, 128), jnp.float32),
    ],
)


def kernel(x, edge_index, batch, W1_rel, W1_root, b1,
           W2_rel, W2_root, b2, W3_rel, W3_root, b3):
    src = edge_index[0].reshape(_NW, _NSTG, _WIN, _B)
    dst = edge_index[1].reshape(_NW, _NSTG, _WIN, _B)
    zeros = jnp.zeros((_N, _C), jnp.float32)
    _agg = _get_agg()

    p1 = _agg(x, src, dst, zeros)
    h1 = _layer1(p1, x, W1_rel, W1_root, b1.reshape(1, -1))
    p2 = _agg(h1, src, dst, zeros)
    h2a, h2b = _layer2(p2, h1, W2_rel, W2_root, b2.reshape(1, -1))
    pa = _agg(h2a, src, dst, zeros)
    pb = _agg(h2b, src, dst, zeros)
    out = _l3(pa, pb, h2a, h2b,
              W3_rel[:_C], W3_rel[_C:], W3_root[:_C], W3_root[_C:],
              b3.reshape(1, -1), batch.reshape(_N // _BM, 1, _BM))
    return out
